# Initial kernel scaffold; baseline (speedup 1.0000x reference)
import functools

import jax
import jax.numpy as jnp
from jax import lax
from jax.experimental import pallas as pl
from jax.experimental.pallas import tpu as pltpu
from jax.experimental.pallas import tpu_sc as plsc

N = 10000
E = 320000
D = 128
NC = 2
NS = 16
NW = NC * NS
CHUNK = 128
CHUNKS_PER_W = 79
E_PAD = CHUNKS_PER_W * NW * CHUNK
E_PER_W = CHUNKS_PER_W * CHUNK
AGG_ROWS = N + NS
ZROWS = AGG_ROWS // NS
OROWS = N // NS


def _sc_aggregate(h, zeros, src, dst):
    mesh = plsc.VectorSubcoreMesh(core_axis_name="c", subcore_axis_name="s")

    @functools.partial(
        pl.kernel,
        out_type=jax.ShapeDtypeStruct((NC, N, D), jnp.float32),
        mesh=mesh,
        scratch_types=[
            pltpu.VMEM((CHUNKS_PER_W, CHUNK), jnp.int32),
            pltpu.VMEM((CHUNKS_PER_W, CHUNK), jnp.int32),
            pltpu.VMEM((CHUNK, D), jnp.float32),
            pltpu.VMEM((CHUNK, D), jnp.float32),
            pltpu.VMEM_SHARED((AGG_ROWS, D), jnp.float32),
            pltpu.SemaphoreType.DMA,
            pltpu.SemaphoreType.DMA,
        ],
    )
    def agg_kernel(h_hbm, z_hbm, src_hbm, dst_hbm, out_hbm,
                   src_v, dst_v, rows0, rows1, acc_sh, sem0, sem1):
        c = lax.axis_index("c")
        s = lax.axis_index("s")
        wid = c * NS + s

        pltpu.sync_copy(z_hbm.at[pl.ds(s * ZROWS, ZROWS)],
                        acc_sh.at[pl.ds(s * ZROWS, ZROWS)])
        pltpu.sync_copy(src_hbm.at[wid], src_v)
        pltpu.sync_copy(dst_hbm.at[wid], dst_v)
        plsc.subcore_barrier()

        bufs = (rows0, rows1)
        sems = (sem0, sem1)

        def gather(k, slot):
            return pltpu.async_copy(h_hbm.at[src_v.at[k]], bufs[slot], sems[slot])

        gather(0, 0)

        def body(k, carry):
            slot = lax.rem(k, 2)

            @pl.when(k + 1 < CHUNKS_PER_W)
            def _():
                gather(k + 1, 1 - slot)

            @pl.when(slot == 0)
            def _():
                pltpu.make_async_copy(h_hbm.at[src_v.at[k]], rows0, sem0).wait()
                pltpu.sync_copy(rows0, acc_sh.at[dst_v.at[k]], add=True)

            @pl.when(slot == 1)
            def _():
                pltpu.make_async_copy(h_hbm.at[src_v.at[k]], rows1, sem1).wait()
                pltpu.sync_copy(rows1, acc_sh.at[dst_v.at[k]], add=True)

            return carry

        lax.fori_loop(0, CHUNKS_PER_W, body, 0)
        plsc.subcore_barrier()

        pltpu.sync_copy(acc_sh.at[pl.ds(s * OROWS, OROWS)],
                        out_hbm.at[c, pl.ds(s * OROWS, OROWS)])

    return agg_kernel(h, zeros, src, dst)


def _tc_linear(parts, w_t, b, relu):
    blk = 1000
    grid = N // blk

    def body(*refs):
        *a_refs, w_ref, b_ref, o_ref = refs
        h = a_refs[0][...]
        for r in a_refs[1:]:
            h = h + r[...]
        y = jnp.dot(h, w_ref[...], preferred_element_type=jnp.float32) + b_ref[...]
        if relu:
            y = jnp.maximum(y, 0.0)
        o_ref[...] = y

    in_specs = [pl.BlockSpec((blk, D), lambda i: (i, 0)) for _ in parts]
    in_specs += [
        pl.BlockSpec((D, D), lambda i: (0, 0)),
        pl.BlockSpec((1, D), lambda i: (0, 0)),
    ]
    return pl.pallas_call(
        body,
        grid=(grid,),
        in_specs=in_specs,
        out_specs=pl.BlockSpec((blk, D), lambda i: (i, 0)),
        out_shape=jax.ShapeDtypeStruct((N, D), jnp.float32),
    )(*parts, w_t, b)


def kernel(x, edge_index, W_conv, b_conv, W_out, b_out):
    ei = edge_index.astype(jnp.int32)
    pad = E_PAD - E
    src = jnp.concatenate([ei[0], jnp.zeros((pad,), jnp.int32)])
    dst = jnp.concatenate([ei[1], jnp.full((pad,), N, jnp.int32)])
    src = src.reshape(NW, CHUNKS_PER_W, CHUNK)
    dst = dst.reshape(NW, CHUNKS_PER_W, CHUNK)
    zeros = jnp.zeros((AGG_ROWS, D), jnp.float32)
    wc_t = W_conv.T
    wo_t = W_out.T
    bc = b_conv.reshape(1, D)
    bo = b_out.reshape(1, D)

    h = x
    for _ in range(2):
        parts = _sc_aggregate(h, zeros, src, dst)
        h = _tc_linear((parts[0], parts[1]), wc_t, bc, relu=True)
    return _tc_linear((h,), wo_t, bo, relu=False)


# speedup vs baseline: 5.5281x; 5.5281x over previous
import functools

import jax
import jax.numpy as jnp
from jax import lax
from jax.experimental import pallas as pl
from jax.experimental.pallas import tpu as pltpu
from jax.experimental.pallas import tpu_sc as plsc

N = 10000
E = 320000
D = 128
NC = 2
NS = 16
NW = NC * NS
CHUNK = 128
CHUNKS_PER_W = 79
E_PAD = CHUNKS_PER_W * NW * CHUNK
E_PER_W = CHUNKS_PER_W * CHUNK
AGG_ROWS = 10016
ZROWS = 632
ZROWS_LAST = AGG_ROWS - (NS - 1) * ZROWS
OROWS = 632
OROWS_LAST = N - (NS - 1) * OROWS


def _sc_aggregate(h, zeros, src, dst):
    mesh = plsc.VectorSubcoreMesh(core_axis_name="c", subcore_axis_name="s",
                                  num_cores=NC, num_subcores=NS)

    @functools.partial(
        pl.kernel,
        out_type=jax.ShapeDtypeStruct((NC, N, D), jnp.float32),
        mesh=mesh,
        scratch_types=[
            pltpu.VMEM((CHUNKS_PER_W, CHUNK), jnp.int32),
            pltpu.VMEM((CHUNK,), jnp.int32),
            pltpu.VMEM((CHUNK,), jnp.int32),
            pltpu.VMEM((CHUNK, D), jnp.float32),
            pltpu.VMEM((CHUNK, D), jnp.float32),
            pltpu.SemaphoreType.DMA,
            pltpu.SemaphoreType.DMA,
            pltpu.SemaphoreType.DMA,
            pltpu.SemaphoreType.DMA,
            pltpu.VMEM_SHARED((AGG_ROWS, D), jnp.float32),
        ],
    )
    def agg_kernel(h_hbm, z_hbm, src_hbm, dst_hbm, out_hbm,
                   src_v, dst0, dst1, rows0, rows1,
                   gsem0, gsem1, dsem0, dsem1, acc_sh):
        c = lax.axis_index("c")
        s = lax.axis_index("s")
        wid = c * NS + s

        @pl.when(s < NS - 1)
        def _():
            pltpu.sync_copy(z_hbm.at[pl.ds(0, ZROWS)],
                            acc_sh.at[pl.ds(s * ZROWS, ZROWS)])

        @pl.when(s == NS - 1)
        def _():
            pltpu.sync_copy(z_hbm.at[pl.ds(0, ZROWS_LAST)],
                            acc_sh.at[pl.ds((NS - 1) * ZROWS, ZROWS_LAST)])

        pltpu.sync_copy(src_hbm.at[wid], src_v)
        plsc.subcore_barrier()

        pltpu.async_copy(dst_hbm.at[wid, 0], dst0, dsem0)
        pltpu.async_copy(h_hbm.at[src_v.at[0]], rows0, gsem0)

        def step(k, dst_a, dsem_a, rows_a, gsem_a, dst_b, dsem_b, rows_b, gsem_b):
            @pl.when(k + 1 < CHUNKS_PER_W)
            def _():
                pltpu.async_copy(dst_hbm.at[wid, k + 1], dst_b, dsem_b)
                pltpu.async_copy(h_hbm.at[src_v.at[k + 1]], rows_b, gsem_b)

            pltpu.make_async_copy(dst_hbm.at[wid, k], dst_a, dsem_a).wait()
            pltpu.make_async_copy(h_hbm.at[src_v.at[k]], rows_a, gsem_a).wait()
            pltpu.sync_copy(rows_a, acc_sh.at[dst_a], add=True)

        def body(k, carry):
            @pl.when(lax.rem(k, 2) == 0)
            def _():
                step(k, dst0, dsem0, rows0, gsem0, dst1, dsem1, rows1, gsem1)

            @pl.when(lax.rem(k, 2) == 1)
            def _():
                step(k, dst1, dsem1, rows1, gsem1, dst0, dsem0, rows0, gsem0)

            return carry

        lax.fori_loop(0, CHUNKS_PER_W, body, 0)
        plsc.subcore_barrier()

        @pl.when(s < NS - 1)
        def _():
            pltpu.sync_copy(acc_sh.at[pl.ds(s * OROWS, OROWS)],
                            out_hbm.at[c, pl.ds(s * OROWS, OROWS)])

        @pl.when(s == NS - 1)
        def _():
            pltpu.sync_copy(acc_sh.at[pl.ds((NS - 1) * OROWS, OROWS_LAST)],
                            out_hbm.at[c, pl.ds((NS - 1) * OROWS, OROWS_LAST)])

    return agg_kernel(h, zeros, src, dst)


def _tc_linear(parts, w_t, b, relu):
    blk = 1000
    grid = N // blk

    def body(*refs):
        *a_refs, w_ref, b_ref, o_ref = refs
        h = a_refs[0][...]
        for r in a_refs[1:]:
            h = h + r[...]
        y = jnp.dot(h, w_ref[...], preferred_element_type=jnp.float32) + b_ref[...]
        if relu:
            y = jnp.maximum(y, 0.0)
        o_ref[...] = y

    in_specs = [pl.BlockSpec((blk, D), lambda i: (i, 0)) for _ in parts]
    in_specs += [
        pl.BlockSpec((D, D), lambda i: (0, 0)),
        pl.BlockSpec((1, D), lambda i: (0, 0)),
    ]
    return pl.pallas_call(
        body,
        grid=(grid,),
        in_specs=in_specs,
        out_specs=pl.BlockSpec((blk, D), lambda i: (i, 0)),
        out_shape=jax.ShapeDtypeStruct((N, D), jnp.float32),
    )(*parts, w_t, b)


def kernel(x, edge_index, W_conv, b_conv, W_out, b_out):
    ei = edge_index.astype(jnp.int32)
    pad = E_PAD - E
    src = jnp.concatenate([ei[0], jnp.zeros((pad,), jnp.int32)])
    dst = jnp.concatenate([ei[1], jnp.full((pad,), N, jnp.int32)])
    src = src.reshape(NW, CHUNKS_PER_W, CHUNK)
    dst = dst.reshape(NW, CHUNKS_PER_W, CHUNK)
    zeros = jnp.zeros((ZROWS, D), jnp.float32)
    wc_t = W_conv.T
    wo_t = W_out.T
    bc = b_conv.reshape(1, D)
    bo = b_out.reshape(1, D)

    h = x
    for _ in range(2):
        parts = _sc_aggregate(h, zeros, src, dst)
        h = _tc_linear((parts[0], parts[1]), wc_t, bc, relu=True)
    return _tc_linear((h,), wo_t, bo, relu=False)
